# baseline (speedup 1.0000x reference)
import functools

import jax
import jax.numpy as jnp
from jax import lax
from jax.experimental import pallas as pl
from jax.experimental.pallas import tpu as pltpu
from jax.experimental.pallas import tpu_sc as plsc

B = 8
H = W = 512
P = 16
N = 1024
PD = 768
D = 256
C = 80
K = 300
KP = 304

_MEAN = (123.675, 116.28, 103.53)
_STD = (58.395, 57.12, 57.375)


def _sc_patchify(x):
    x6 = x.reshape(B, 3, H // P, P, W // P, P)
    mesh = plsc.VectorSubcoreMesh(core_axis_name="c", subcore_axis_name="s")

    @functools.partial(
        pl.kernel,
        mesh=mesh,
        compiler_params=pltpu.CompilerParams(needs_layout_passes=False),
        out_type=jax.ShapeDtypeStruct((B, H // P, W // P, 48, P),
                                      jnp.float32),
        scratch_types=[pltpu.VMEM((H // P, W // P, P), jnp.float32)],
    )
    def k(x_hbm, p_hbm, buf):
        cid = lax.axis_index("c")
        sid = lax.axis_index("s")
        wid = sid * 2 + cid
        for t in range(12):
            pair = wid * 12 + t
            b = pair // 48
            ci = pair % 48
            c = 2 - ci // P
            i = ci % P
            pltpu.sync_copy(x_hbm.at[b, c, :, i, :, :], buf)
            pltpu.sync_copy(buf, p_hbm.at[b, :, :, ci, :])

    return k(x6).reshape(B, N, PD)


def _tc_body(p_ref, mean_ref, std_ref, we_ref, be_ref, wb_ref, bb_ref,
             wc_ref, bc_ref, vals_ref, rank_ref):
    pn = (p_ref[0] - mean_ref[...]) / std_ref[...]
    feats = jnp.tanh(
        jnp.dot(pn, we_ref[...], preferred_element_type=jnp.float32)
        + be_ref[...])
    logits = (jnp.dot(feats, wc_ref[...], preferred_element_type=jnp.float32)
              + bc_ref[...])
    probs = jax.nn.sigmoid(logits)
    score = jnp.max(probs, axis=1, keepdims=True)
    cls_iota = lax.broadcasted_iota(jnp.int32, (1, C), 1)
    label = jnp.min(jnp.where(probs == score, cls_iota, C), axis=1,
                    keepdims=True)
    boxes = jax.nn.sigmoid(
        jnp.dot(feats, wb_ref[...], preferred_element_type=jnp.float32)
        + bb_ref[...])
    cols = jnp.concatenate(
        [boxes, score, label.astype(jnp.float32),
         jnp.zeros((N, 2), jnp.float32)], axis=1)
    colsT = cols.T
    vals_ref[0] = colsT

    srow = colsT[4:5]
    irow = lax.broadcasted_iota(jnp.int32, (1, N), 1)
    rank = jnp.zeros((1, N), jnp.int32)
    for jc in range(4):
        scol = score[jc * 256:(jc + 1) * 256]
        icol = (lax.broadcasted_iota(jnp.int32, (256, 1), 0) + jc * 256)
        gt = scol > srow
        tie = (scol == srow) & (icol < irow)
        rank = rank + jnp.sum((gt | tie).astype(jnp.int32), axis=0,
                              keepdims=True)
    rank_ref[0] = rank


def _tc_call(p, mean_v, std_v, We, be, Wb, bb, Wc, bc):
    return pl.pallas_call(
        _tc_body,
        grid=(B,),
        in_specs=[
            pl.BlockSpec((1, N, PD), lambda i: (i, 0, 0)),
            pl.BlockSpec((1, PD), lambda i: (0, 0)),
            pl.BlockSpec((1, PD), lambda i: (0, 0)),
            pl.BlockSpec((PD, D), lambda i: (0, 0)),
            pl.BlockSpec((1, D), lambda i: (0, 0)),
            pl.BlockSpec((D, 4), lambda i: (0, 0)),
            pl.BlockSpec((1, 4), lambda i: (0, 0)),
            pl.BlockSpec((D, C), lambda i: (0, 0)),
            pl.BlockSpec((1, C), lambda i: (0, 0)),
        ],
        out_specs=[
            pl.BlockSpec((1, 8, N), lambda i: (i, 0, 0)),
            pl.BlockSpec((1, 1, N), lambda i: (i, 0, 0)),
        ],
        out_shape=[
            jax.ShapeDtypeStruct((B, 8, N), jnp.float32),
            jax.ShapeDtypeStruct((B, 1, N), jnp.int32),
        ],
    )(p, mean_v, std_v, We, be, Wb, bb, Wc, bc)


def _sc_call(vals, rank):
    mesh = plsc.VectorSubcoreMesh(core_axis_name="c", subcore_axis_name="s")

    @functools.partial(
        pl.kernel,
        mesh=mesh,
        compiler_params=pltpu.CompilerParams(needs_layout_passes=False),
        out_type=jax.ShapeDtypeStruct((B, 6, KP), jnp.float32),
        scratch_types=[
            pltpu.VMEM((N,), jnp.int32),
            pltpu.VMEM((64, 128), jnp.float32),
            pltpu.VMEM((8, 128), jnp.int32),
            pltpu.VMEM((6, KP), jnp.float32),
        ],
    )
    def k(vals_hbm, rank_hbm, out_hbm, rank_v, vals_v, inv_v, out_v):
        cid = lax.axis_index("c")
        sid = lax.axis_index("s")
        wid = sid * 2 + cid

        @pl.when(wid < B)
        def _():
            pltpu.sync_copy(rank_hbm.at[wid], rank_v)
            pltpu.sync_copy(vals_hbm.at[wid], vals_v)
            for g in range(N // 16):
                rv = rank_v[pl.ds(g * 16, 16)]
                iv = lax.iota(jnp.int32, 16) + g * 16
                plsc.store_scatter(inv_v, [rv >> 7, rv & 127], iv)
            for rg in range(KP // 16):
                src = inv_v[rg >> 3, pl.ds((rg & 7) * 16, 16)]
                for c in range(6):
                    out_v[c, pl.ds(rg * 16, 16)] = plsc.load_gather(
                        vals_v, [(c * 8) + (src >> 7), src & 127])
            pltpu.sync_copy(out_v, out_hbm.at[wid])

    return k(vals, rank)


def kernel(x, W_embed, b_embed, W_box, b_box, W_cls, b_cls):
    mean_v = jnp.repeat(jnp.asarray(_MEAN, jnp.float32), P * P).reshape(1, PD)
    std_v = jnp.repeat(jnp.asarray(_STD, jnp.float32), P * P).reshape(1, PD)
    xp = _sc_patchify(x)
    vals, rank = _tc_call(xp, mean_v, std_v, W_embed,
                          b_embed.reshape(1, D), W_box, b_box.reshape(1, 4),
                          W_cls, b_cls.reshape(1, C))
    out = _sc_call(vals.reshape(B, 64, 128), rank.reshape(B, N))
    return out[:, :, :K].transpose(0, 2, 1)


# speedup vs baseline: 1.9650x; 1.9650x over previous
import functools

import jax
import jax.numpy as jnp
from jax import lax
from jax.experimental import pallas as pl
from jax.experimental.pallas import tpu as pltpu
from jax.experimental.pallas import tpu_sc as plsc

B = 8
H = W = 512
P = 16
N = 1024
PD = 768
D = 256
C = 80
K = 300
KP = 304

_MEAN = (123.675, 116.28, 103.53)
_STD = (58.395, 57.12, 57.375)


def _tc_body(*refs):
    pieces = refs[:48]
    (mean_ref, std_ref, we_ref, be_ref, wb_ref, bb_ref, wc_ref, bc_ref,
     vals_ref, rank_ref) = refs[48:]
    p = jnp.concatenate(
        [r[0, 0, :, 0].reshape(N, P) for r in pieces], axis=1)
    pn = (p - mean_ref[...]) / std_ref[...]
    feats = jnp.tanh(
        jnp.dot(pn, we_ref[...], preferred_element_type=jnp.float32)
        + be_ref[...])
    logits = (jnp.dot(feats, wc_ref[...], preferred_element_type=jnp.float32)
              + bc_ref[...])
    probs = jax.nn.sigmoid(logits)
    score = jnp.max(probs, axis=1, keepdims=True)
    cls_iota = lax.broadcasted_iota(jnp.int32, (1, C), 1)
    label = jnp.min(jnp.where(probs == score, cls_iota, C), axis=1,
                    keepdims=True)
    boxes = jax.nn.sigmoid(
        jnp.dot(feats, wb_ref[...], preferred_element_type=jnp.float32)
        + bb_ref[...])
    cols = jnp.concatenate(
        [boxes, score, label.astype(jnp.float32),
         jnp.zeros((N, 2), jnp.float32)], axis=1)
    colsT = cols.T
    vals_ref[0] = colsT

    srow = colsT[4:5]
    irow = lax.broadcasted_iota(jnp.int32, (1, N), 1)
    rank = jnp.zeros((1, N), jnp.int32)
    for jc in range(4):
        scol = score[jc * 256:(jc + 1) * 256]
        icol = (lax.broadcasted_iota(jnp.int32, (256, 1), 0) + jc * 256)
        gt = scol > srow
        tie = (scol == srow) & (icol < irow)
        rank = rank + jnp.sum((gt | tie).astype(jnp.int32), axis=0,
                              keepdims=True)
    rank_ref[0] = rank


def _tc_call(x, mean_v, std_v, We, be, Wb, bb, Wc, bc):
    x6 = x.reshape(B, 3, H // P, P, W // P, P)
    piece_specs = [
        pl.BlockSpec((1, 1, H // P, 1, W // P, P),
                     lambda b, _c=2 - ci // P, _i=ci % P: (b, _c, 0, _i, 0, 0))
        for ci in range(48)
    ]
    return pl.pallas_call(
        _tc_body,
        grid=(B,),
        in_specs=piece_specs + [
            pl.BlockSpec((1, PD), lambda i: (0, 0)),
            pl.BlockSpec((1, PD), lambda i: (0, 0)),
            pl.BlockSpec((PD, D), lambda i: (0, 0)),
            pl.BlockSpec((1, D), lambda i: (0, 0)),
            pl.BlockSpec((D, 4), lambda i: (0, 0)),
            pl.BlockSpec((1, 4), lambda i: (0, 0)),
            pl.BlockSpec((D, C), lambda i: (0, 0)),
            pl.BlockSpec((1, C), lambda i: (0, 0)),
        ],
        out_specs=[
            pl.BlockSpec((1, 8, N), lambda i: (i, 0, 0)),
            pl.BlockSpec((1, 1, N), lambda i: (i, 0, 0)),
        ],
        out_shape=[
            jax.ShapeDtypeStruct((B, 8, N), jnp.float32),
            jax.ShapeDtypeStruct((B, 1, N), jnp.int32),
        ],
    )(*([x6] * 48), mean_v, std_v, We, be, Wb, bb, Wc, bc)


def _sc_call(vals, rank):
    mesh = plsc.VectorSubcoreMesh(core_axis_name="c", subcore_axis_name="s")

    @functools.partial(
        pl.kernel,
        mesh=mesh,
        compiler_params=pltpu.CompilerParams(needs_layout_passes=False),
        out_type=jax.ShapeDtypeStruct((B, 6, KP), jnp.float32),
        scratch_types=[
            pltpu.VMEM((N,), jnp.int32),
            pltpu.VMEM((64, 128), jnp.float32),
            pltpu.VMEM((8, 128), jnp.int32),
            pltpu.VMEM((6, KP), jnp.float32),
        ],
    )
    def k(vals_hbm, rank_hbm, out_hbm, rank_v, vals_v, inv_v, out_v):
        cid = lax.axis_index("c")
        sid = lax.axis_index("s")
        wid = sid * 2 + cid

        @pl.when(wid < B)
        def _():
            pltpu.sync_copy(rank_hbm.at[wid], rank_v)
            pltpu.sync_copy(vals_hbm.at[wid], vals_v)
            for g in range(N // 16):
                rv = rank_v[pl.ds(g * 16, 16)]
                iv = lax.iota(jnp.int32, 16) + g * 16
                plsc.store_scatter(inv_v, [rv >> 7, rv & 127], iv)
            for rg in range(KP // 16):
                src = inv_v[rg >> 3, pl.ds((rg & 7) * 16, 16)]
                for c in range(6):
                    out_v[c, pl.ds(rg * 16, 16)] = plsc.load_gather(
                        vals_v, [(c * 8) + (src >> 7), src & 127])
            pltpu.sync_copy(out_v, out_hbm.at[wid])

    return k(vals, rank)


def kernel(x, W_embed, b_embed, W_box, b_box, W_cls, b_cls):
    mean_v = jnp.repeat(jnp.asarray(_MEAN, jnp.float32), P * P).reshape(1, PD)
    std_v = jnp.repeat(jnp.asarray(_STD, jnp.float32), P * P).reshape(1, PD)
    vals, rank = _tc_call(x, mean_v, std_v, W_embed,
                          b_embed.reshape(1, D), W_box, b_box.reshape(1, 4),
                          W_cls, b_cls.reshape(1, C))
    out = _sc_call(vals.reshape(B, 64, 128), rank.reshape(B, N))
    return out[:, :, :K].transpose(0, 2, 1)


# speedup vs baseline: 2.4103x; 1.2266x over previous
import functools

import jax
import jax.numpy as jnp
from jax import lax
from jax.experimental import pallas as pl
from jax.experimental.pallas import tpu as pltpu
from jax.experimental.pallas import tpu_sc as plsc

B = 8
H = W = 512
P = 16
N = 1024
PD = 768
D = 256
C = 80
K = 300
KP = 304

_MEAN = (123.675, 116.28, 103.53)
_STD = (58.395, 57.12, 57.375)


IPB = 1


def _tc_body(x_ref, mean_ref, std_ref, we_ref, be_ref, wb_ref, bb_ref,
             wc_ref, bc_ref, vals_ref, rank_ref):
    x0 = x_ref[0]
    xf = jnp.concatenate([x0[2:3], x0[1:2], x0[0:1]], axis=0)
    p = (xf.reshape(3, H // P, P, W // P, P)
         .transpose(1, 3, 0, 2, 4)
         .reshape(N, PD))
    pn = (p - mean_ref[...]) / std_ref[...]
    feats = jnp.tanh(
        jnp.dot(pn, we_ref[...], preferred_element_type=jnp.float32)
        + be_ref[...])
    logits = (jnp.dot(feats, wc_ref[...], preferred_element_type=jnp.float32)
              + bc_ref[...])
    probs = jax.nn.sigmoid(logits)
    score = jnp.max(probs, axis=1, keepdims=True)
    cls_iota = lax.broadcasted_iota(jnp.int32, (1, C), 1)
    label = jnp.min(jnp.where(probs == score, cls_iota, C), axis=1,
                    keepdims=True)
    boxes = jax.nn.sigmoid(
        jnp.dot(feats, wb_ref[...], preferred_element_type=jnp.float32)
        + bb_ref[...])
    cols = jnp.concatenate(
        [boxes, score, label.astype(jnp.float32),
         jnp.zeros((N, 2), jnp.float32)], axis=1)
    colsT = cols.T
    vals_ref[0] = colsT
    srow = colsT[4:5]
    irow = lax.broadcasted_iota(jnp.int32, (1, N), 1)
    rank = jnp.zeros((1, N), jnp.int32)
    for jc in range(4):
        scol = score[jc * 256:(jc + 1) * 256]
        icol = (lax.broadcasted_iota(jnp.int32, (256, 1), 0) + jc * 256)
        gt = scol > srow
        tie = (scol == srow) & (icol < irow)
        rank = rank + jnp.sum((gt | tie).astype(jnp.int32), axis=0,
                              keepdims=True)
    rank_ref[0] = rank


def _tc_call(x, mean_v, std_v, We, be, Wb, bb, Wc, bc):
    return pl.pallas_call(
        _tc_body,
        grid=(B,),
        in_specs=[
            pl.BlockSpec((1, 3, H, W), lambda i: (i, 0, 0, 0)),
            pl.BlockSpec((1, PD), lambda i: (0, 0)),
            pl.BlockSpec((1, PD), lambda i: (0, 0)),
            pl.BlockSpec((PD, D), lambda i: (0, 0)),
            pl.BlockSpec((1, D), lambda i: (0, 0)),
            pl.BlockSpec((D, 4), lambda i: (0, 0)),
            pl.BlockSpec((1, 4), lambda i: (0, 0)),
            pl.BlockSpec((D, C), lambda i: (0, 0)),
            pl.BlockSpec((1, C), lambda i: (0, 0)),
        ],
        out_specs=[
            pl.BlockSpec((1, 8, N), lambda i: (i, 0, 0)),
            pl.BlockSpec((1, 1, N), lambda i: (i, 0, 0)),
        ],
        out_shape=[
            jax.ShapeDtypeStruct((B, 8, N), jnp.float32),
            jax.ShapeDtypeStruct((B, 1, N), jnp.int32),
        ],
    )(x, mean_v, std_v, We, be, Wb, bb, Wc, bc)


def _sc_call(vals, rank):
    mesh = plsc.VectorSubcoreMesh(core_axis_name="c", subcore_axis_name="s")

    @functools.partial(
        pl.kernel,
        mesh=mesh,
        compiler_params=pltpu.CompilerParams(needs_layout_passes=False),
        out_type=jax.ShapeDtypeStruct((B, 6, KP), jnp.float32),
        scratch_types=[
            pltpu.VMEM((N,), jnp.int32),
            pltpu.VMEM((64, 128), jnp.float32),
            pltpu.VMEM((8, 128), jnp.int32),
            pltpu.VMEM((6, KP), jnp.float32),
        ],
    )
    def k(vals_hbm, rank_hbm, out_hbm, rank_v, vals_v, inv_v, out_v):
        cid = lax.axis_index("c")
        sid = lax.axis_index("s")
        wid = sid * 2 + cid

        @pl.when(wid < B)
        def _():
            pltpu.sync_copy(rank_hbm.at[wid], rank_v)
            pltpu.sync_copy(vals_hbm.at[wid], vals_v)
            for g in range(N // 16):
                rv = rank_v[pl.ds(g * 16, 16)]
                iv = lax.iota(jnp.int32, 16) + g * 16
                plsc.store_scatter(inv_v, [rv >> 7, rv & 127], iv)
            for rg in range(KP // 16):
                src = inv_v[rg >> 3, pl.ds((rg & 7) * 16, 16)]
                for c in range(6):
                    out_v[c, pl.ds(rg * 16, 16)] = plsc.load_gather(
                        vals_v, [(c * 8) + (src >> 7), src & 127])
            pltpu.sync_copy(out_v, out_hbm.at[wid])

    return k(vals, rank)


def kernel(x, W_embed, b_embed, W_box, b_box, W_cls, b_cls):
    mean_v = jnp.repeat(jnp.asarray(_MEAN, jnp.float32), P * P).reshape(1, PD)
    std_v = jnp.repeat(jnp.asarray(_STD, jnp.float32), P * P).reshape(1, PD)
    vals, rank = _tc_call(x, mean_v, std_v, W_embed,
                          b_embed.reshape(1, D), W_box, b_box.reshape(1, 4),
                          W_cls, b_cls.reshape(1, C))
    out = _sc_call(vals.reshape(B, 64, 128), rank.reshape(B, N))
    return out[:, :, :K].transpose(0, 2, 1)
